# Initial kernel scaffold; baseline (speedup 1.0000x reference)
import jax
import jax.numpy as jnp
from jax.experimental import pallas as pl


def kernel(x, edge_index, W_np, b_np, W_g0, b_g0, W_g1, b_g1, W_ih, W_hh, b_ih, b_hh, W_fc, b_fc):
    raise NotImplementedError("write your pallas kernel here")


# speedup vs baseline: 5.0519x; 5.0519x over previous
import jax
import jax.numpy as jnp
from jax.experimental import pallas as pl
from jax.experimental.pallas import tpu as pltpu

B, T, N, D = 32, 1024, 14, 3
H = 64
ND = N * D
NH = N * H
G3 = 3 * H
TCH = 64
NCHUNK = T // TCH
R = B * TCH

F32 = jnp.float32


def _dotT(a, b):
    return jax.lax.dot_general(a, b, (((1,), (1,)), ((), ())),
                               preferred_element_type=F32)


def _dot(a, b):
    return jax.lax.dot_general(a, b, (((1,), (0,)), ((), ())),
                               preferred_element_type=F32)


def _gnn_gru_kernel(x_ref, ei_ref, Wnp_ref, bnp_ref, Wg0_ref, bg0_ref,
                    Wg1_ref, bg1_ref, Wih_ref, Whh_ref, bih_ref, bhh_ref,
                    Wfc_ref, bfc_ref, out_ref,
                    A0_s, A1_s, Wbig_s, brow_s, gi_s, h_s):
    pid = pl.program_id(0)

    @pl.when(pid == 0)
    def _prep():
        i896 = jax.lax.broadcasted_iota(jnp.int32, (NH, 1), 0)
        or896 = (i896 // H == jax.lax.broadcasted_iota(
            jnp.int32, (NH, N), 1)).astype(F32)
        oh_h = (i896 % H == jax.lax.broadcasted_iota(
            jnp.int32, (NH, H), 1)).astype(F32)

        lanesN = jax.lax.broadcasted_iota(jnp.int32, (N, N), 1)
        ohi = (ei_ref[:, 0:1] == lanesN).astype(F32)
        ohj = (ei_ref[:, 1:2] == lanesN).astype(F32)
        M = (jax.lax.dot_general(ohi, ohj, (((0,), (0,)), ((), ())),
                                 preferred_element_type=F32) +
             jax.lax.dot_general(ohj, ohi, (((0,), (0,)), ((), ())),
                                 preferred_element_type=F32))

        Pexp = _dotT(_dot(or896, M), or896)
        Q0 = _dotT(_dotT(oh_h, Wg0_ref[...]), oh_h)
        Q1 = _dotT(_dotT(oh_h, Wg1_ref[...]), oh_h)
        A0_s[...] = Pexp * Q0
        A1_s[...] = Pexp * Q1

        i42 = jax.lax.broadcasted_iota(jnp.int32, (ND, 1), 0)
        eq_nm = (i42 // D == jax.lax.broadcasted_iota(
            jnp.int32, (ND, NH), 1) // H).astype(F32)
        oh_d = (i42 % D == jax.lax.broadcasted_iota(
            jnp.int32, (ND, D), 1)).astype(F32)
        Wbig_s[...] = eq_nm * _dotT(_dotT(oh_d, Wnp_ref[...]), oh_h)

        brow_s[0:1, :] = _dotT(bnp_ref[...], oh_h)
        rs = _dotT(jnp.sum(M, axis=0, keepdims=True), or896)
        brow_s[1:2, :] = rs * _dotT(bg0_ref[...], oh_h)
        brow_s[2:3, :] = rs * _dotT(bg1_ref[...], oh_h)

        h_s[...] = jnp.zeros((B, H), F32)

    xf = x_ref[...].reshape(R, ND)
    h0 = _dot(xf, Wbig_s[...]) + brow_s[0:1, :]
    h1 = jnp.maximum(_dot(h0, A0_s[...]) + brow_s[1:2, :], 0.0)
    h2 = jnp.maximum(_dot(h1, A1_s[...]) + brow_s[2:3, :], 0.0)
    lanes = jax.lax.broadcasted_iota(jnp.int32, (1, G3), 1)
    gbias = bih_ref[...] + jnp.where(lanes < 2 * H, bhh_ref[...], 0.0)
    gi_s[...] = (_dotT(h2, Wih_ref[...]) + gbias).reshape(B, TCH, G3)

    bhh_n = bhh_ref[0:1, 2 * H:]
    Whh = Whh_ref[...]

    def step(t, h):
        g = gi_s[:, pl.ds(t, 1), :].reshape(B, G3)
        gh = _dotT(h, Whh)
        r = jax.nn.sigmoid(g[:, :H] + gh[:, :H])
        z = jax.nn.sigmoid(g[:, H:2 * H] + gh[:, H:2 * H])
        n = jnp.tanh(g[:, 2 * H:] + r * (gh[:, 2 * H:] + bhh_n))
        return (1.0 - z) * n + z * h

    h_fin = jax.lax.fori_loop(0, TCH, step, h_s[...])
    h_s[...] = h_fin

    @pl.when(pid == NCHUNK - 1)
    def _fin():
        Wfc_b = jnp.broadcast_to(Wfc_ref[...], (128, H))
        out_ref[...] = _dotT(h_fin, Wfc_b) + bfc_ref[...]


def kernel(x, edge_index, W_np, b_np, W_g0, b_g0, W_g1, b_g1,
           W_ih, W_hh, b_ih, b_hh, W_fc, b_fc):
    xr = x.reshape(B, T, ND)
    full = lambda s: pl.BlockSpec(s, lambda i: (0,) * len(s))
    res = pl.pallas_call(
        _gnn_gru_kernel,
        grid=(NCHUNK,),
        in_specs=[
            pl.BlockSpec((B, TCH, ND), lambda i: (0, i, 0)),
            full((N, 2)),
            full((H, D)), full((1, H)),
            full((H, H)), full((1, H)),
            full((H, H)), full((1, H)),
            full((G3, NH)), full((G3, H)),
            full((1, G3)), full((1, G3)),
            full((1, H)), full((1, 128)),
        ],
        out_specs=pl.BlockSpec((B, 128), lambda i: (0, 0)),
        out_shape=jax.ShapeDtypeStruct((B, 128), F32),
        scratch_shapes=[
            pltpu.VMEM((NH, NH), F32),
            pltpu.VMEM((NH, NH), F32),
            pltpu.VMEM((ND, NH), F32),
            pltpu.VMEM((3, NH), F32),
            pltpu.VMEM((B, TCH, G3), F32),
            pltpu.VMEM((B, H), F32),
        ],
    )(xr, edge_index, W_np, b_np.reshape(1, H), W_g0, b_g0.reshape(1, H),
      W_g1, b_g1.reshape(1, H), W_ih, W_hh, b_ih.reshape(1, G3),
      b_hh.reshape(1, G3), W_fc, jnp.broadcast_to(b_fc.reshape(1, 1), (1, 128)))
    return res[:, :1]


# speedup vs baseline: 5.8609x; 1.1601x over previous
import jax
import jax.numpy as jnp
from jax.experimental import pallas as pl
from jax.experimental.pallas import tpu as pltpu

B, T, N, D = 32, 1024, 14, 3
H = 64
ND = N * D
NH = N * H
G3 = 3 * H
TCH = 64
NCHUNK = T // TCH
R = B * TCH

F32 = jnp.float32


def _dotT(a, b):
    return jax.lax.dot_general(a, b, (((1,), (1,)), ((), ())),
                               preferred_element_type=F32)


def _dot(a, b):
    return jax.lax.dot_general(a, b, (((1,), (0,)), ((), ())),
                               preferred_element_type=F32)


def _gnn_gru_kernel(x_ref, ei_ref, Wnp_ref, bnp_ref, Wg0_ref, bg0_ref,
                    Wg1_ref, bg1_ref, Wih_ref, Whh_ref, bih_ref, bhh_ref,
                    Wfc_ref, bfc_ref, out_ref,
                    A0_s, A1_s, Wbig_s, brow_s, gi_s, h_s):
    pid = pl.program_id(0)

    @pl.when(pid == 0)
    def _prep():
        i896 = jax.lax.broadcasted_iota(jnp.int32, (NH, 1), 0)
        or896 = (i896 // H == jax.lax.broadcasted_iota(
            jnp.int32, (NH, N), 1)).astype(F32)
        oh_h = (i896 % H == jax.lax.broadcasted_iota(
            jnp.int32, (NH, H), 1)).astype(F32)

        lanesN = jax.lax.broadcasted_iota(jnp.int32, (N, N), 1)
        ohi = (ei_ref[:, 0:1] == lanesN).astype(F32)
        ohj = (ei_ref[:, 1:2] == lanesN).astype(F32)
        M = (jax.lax.dot_general(ohi, ohj, (((0,), (0,)), ((), ())),
                                 preferred_element_type=F32) +
             jax.lax.dot_general(ohj, ohi, (((0,), (0,)), ((), ())),
                                 preferred_element_type=F32))

        Pexp = _dotT(_dot(or896, M), or896)
        Q0 = _dotT(_dotT(oh_h, Wg0_ref[...]), oh_h)
        Q1 = _dotT(_dotT(oh_h, Wg1_ref[...]), oh_h)
        A0_s[...] = Pexp * Q0
        A1_s[...] = Pexp * Q1

        i42 = jax.lax.broadcasted_iota(jnp.int32, (ND, 1), 0)
        eq_nm = (i42 // D == jax.lax.broadcasted_iota(
            jnp.int32, (ND, NH), 1) // H).astype(F32)
        oh_d = (i42 % D == jax.lax.broadcasted_iota(
            jnp.int32, (ND, D), 1)).astype(F32)
        Wbig_s[...] = eq_nm * _dotT(_dotT(oh_d, Wnp_ref[...]), oh_h)

        brow_s[0:1, :] = _dotT(bnp_ref[...], oh_h)
        rs = _dotT(jnp.sum(M, axis=0, keepdims=True), or896)
        brow_s[1:2, :] = rs * _dotT(bg0_ref[...], oh_h)
        brow_s[2:3, :] = rs * _dotT(bg1_ref[...], oh_h)

        h_s[...] = jnp.zeros((B, H), F32)

    xf = x_ref[...].reshape(R, ND)
    h0 = _dot(xf, Wbig_s[...]) + brow_s[0:1, :]
    h1 = jnp.maximum(_dot(h0, A0_s[...]) + brow_s[1:2, :], 0.0)
    h2 = jnp.maximum(_dot(h1, A1_s[...]) + brow_s[2:3, :], 0.0)
    lanes = jax.lax.broadcasted_iota(jnp.int32, (1, G3), 1)
    gbias = bih_ref[...] + jnp.where(lanes < 2 * H, bhh_ref[...], 0.0)
    gi_s[...] = (_dotT(h2, Wih_ref[...]) + gbias).reshape(B, TCH, G3)

    bhh_n = bhh_ref[0:1, 2 * H:]
    Whh = Whh_ref[...]

    h = h_s[...]
    for t in range(TCH):
        g = gi_s[:, t, :]
        gh = _dotT(h, Whh)
        s_rz = g[:, :2 * H] + gh[:, :2 * H]
        t_rz = jnp.tanh(0.5 * s_rz)
        r = 0.5 + 0.5 * t_rz[:, :H]
        z = 0.5 + 0.5 * t_rz[:, H:]
        n = jnp.tanh(g[:, 2 * H:] + r * (gh[:, 2 * H:] + bhh_n))
        h = n + z * (h - n)
    h_fin = h
    h_s[...] = h_fin

    @pl.when(pid == NCHUNK - 1)
    def _fin():
        Wfc_b = jnp.broadcast_to(Wfc_ref[...], (128, H))
        out_ref[...] = _dotT(h_fin, Wfc_b) + bfc_ref[...]


def kernel(x, edge_index, W_np, b_np, W_g0, b_g0, W_g1, b_g1,
           W_ih, W_hh, b_ih, b_hh, W_fc, b_fc):
    xr = x.reshape(B, T, ND)
    full = lambda s: pl.BlockSpec(s, lambda i: (0,) * len(s))
    res = pl.pallas_call(
        _gnn_gru_kernel,
        grid=(NCHUNK,),
        in_specs=[
            pl.BlockSpec((B, TCH, ND), lambda i: (0, i, 0)),
            full((N, 2)),
            full((H, D)), full((1, H)),
            full((H, H)), full((1, H)),
            full((H, H)), full((1, H)),
            full((G3, NH)), full((G3, H)),
            full((1, G3)), full((1, G3)),
            full((1, H)), full((1, 128)),
        ],
        out_specs=pl.BlockSpec((B, 128), lambda i: (0, 0)),
        out_shape=jax.ShapeDtypeStruct((B, 128), F32),
        scratch_shapes=[
            pltpu.VMEM((NH, NH), F32),
            pltpu.VMEM((NH, NH), F32),
            pltpu.VMEM((ND, NH), F32),
            pltpu.VMEM((3, NH), F32),
            pltpu.VMEM((B, TCH, G3), F32),
            pltpu.VMEM((B, H), F32),
        ],
    )(xr, edge_index, W_np, b_np.reshape(1, H), W_g0, b_g0.reshape(1, H),
      W_g1, b_g1.reshape(1, H), W_ih, W_hh, b_ih.reshape(1, G3),
      b_hh.reshape(1, G3), W_fc, jnp.broadcast_to(b_fc.reshape(1, 1), (1, 128)))
    return res[:, :1]
